# baseline (speedup 1.0000x reference)
import jax
import jax.numpy as jnp
from jax import lax
from jax.experimental import pallas as pl
from jax.experimental.pallas import tpu as pltpu
from jax.experimental.pallas import tpu_sc as plsc

N = 10000
NPAD = 10016
F = 32
ALPHA = 0.1
K_STEPS = 10
NCORES = 2
NSUB = 16
NTILES = NCORES * NSUB
CHUNK = 128
CPT = 79
EPT = CPT * CHUNK
EPAD = EPT * NTILES
ROWS_PT = NPAD // NSUB
NR = NPAD * F // 128

_sc_mesh = plsc.VectorSubcoreMesh(core_axis_name="c", subcore_axis_name="s")


NBUF = 8


def _edge_body(s_hbm, src_hbm, dst_hbm, zeros_hbm, t0_hbm, t1_hbm,
               srcv, dstv, *scr):
    rows = scr[:NBUF]
    tsh = scr[NBUF]
    ssh = scr[NBUF + 1]
    gsems = scr[NBUF + 2:2 * NBUF + 2]
    ssems = scr[2 * NBUF + 2:3 * NBUF + 2]
    zsem = scr[3 * NBUF + 2]
    cid = lax.axis_index("c")
    sid = lax.axis_index("s")
    wid = sid * NCORES + cid
    base = sid * jnp.int32(ROWS_PT)

    zd = pltpu.async_copy(zeros_hbm.at[pl.ds(base, ROWS_PT)],
                          tsh.at[pl.ds(base, ROWS_PT)], zsem)
    pltpu.sync_copy(s_hbm.at[pl.ds(base, ROWS_PT)],
                    ssh.at[pl.ds(base, ROWS_PT)])
    pltpu.sync_copy(src_hbm.at[wid], srcv)
    pltpu.sync_copy(dst_hbm.at[wid], dstv)
    zd.wait()
    plsc.subcore_barrier()

    gdesc = [None] * CPT
    sdesc = [None] * CPT

    def _gather(k):
        gdesc[k] = pltpu.async_copy(
            ssh.at[srcv.at[jnp.int32(k)]], rows[k % NBUF],
            gsems[k % NBUF])

    D = 6
    for k in range(min(D, CPT)):
        _gather(k)
    for j in range(CPT):
        jp = j + D
        if jp < CPT:
            if jp >= NBUF:
                sdesc[jp - NBUF].wait()
            _gather(jp)
        b = j % NBUF
        gdesc[j].wait()
        sdesc[j] = pltpu.async_copy(
            rows[b], tsh.at[dstv.at[jnp.int32(j)]], ssems[b], add=True)
    for j in range(max(CPT - NBUF, 0), CPT):
        sdesc[j].wait()

    plsc.subcore_barrier()

    @pl.when(cid == 0)
    def _w0():
        pltpu.sync_copy(tsh.at[pl.ds(base, ROWS_PT)],
                        t0_hbm.at[pl.ds(base, ROWS_PT)])

    @pl.when(cid == 1)
    def _w1():
        pltpu.sync_copy(tsh.at[pl.ds(base, ROWS_PT)],
                        t1_hbm.at[pl.ds(base, ROWS_PT)])


_edge_kernel = pl.kernel(
    _edge_body,
    out_type=(
        jax.ShapeDtypeStruct((NPAD, F), jnp.float32),
        jax.ShapeDtypeStruct((NPAD, F), jnp.float32),
    ),
    mesh=_sc_mesh,
    compiler_params=pltpu.CompilerParams(use_tc_tiling_on_sc=False),
    scratch_types=(
        [pltpu.VMEM((CPT, CHUNK), jnp.int32),
         pltpu.VMEM((CPT, CHUNK), jnp.int32)]
        + [pltpu.VMEM((CHUNK, F), jnp.float32) for _ in range(NBUF)]
        + [pltpu.VMEM_SHARED((NPAD, F), jnp.float32)]
        + [pltpu.VMEM_SHARED((NPAD, F), jnp.float32)]
        + [pltpu.SemaphoreType.DMA for _ in range(2 * NBUF + 1)]
    ),
)


def _prep_body(x_ref, w1_ref, b1_ref, w2_ref, b2_ref, d0_ref, d1_ref,
               s0_ref, cp_ref, g_ref, c2_ref, g2_ref):
    h1 = jnp.dot(x_ref[...], w1_ref[...], preferred_element_type=jnp.float32)
    h1 = jnp.maximum(h1 + b1_ref[...], 0.0)
    h = jnp.dot(h1, w2_ref[...], preferred_element_type=jnp.float32)
    h = h + b2_ref[...]
    deg = d0_ref[...] + d1_ref[...] + 1.0
    dinv = lax.rsqrt(deg)
    s0 = dinv * h
    s0_ref[...] = s0
    cp_ref[...] = (1.0 - ALPHA) * dinv * dinv
    g_ref[...] = ALPHA * s0
    c2_ref[...] = (1.0 - ALPHA) * dinv
    g2_ref[...] = ALPHA * h


_prep_kernel = pl.pallas_call(
    _prep_body,
    out_shape=tuple(
        jax.ShapeDtypeStruct((NPAD, F), jnp.float32) for _ in range(5)
    ),
)


def _combine_body(t0_ref, t1_ref, s_ref, c_ref, g_ref, out_ref):
    out_ref[...] = (
        c_ref[...] * ((t0_ref[...] + t1_ref[...]) + s_ref[...]) + g_ref[...]
    )


_combine_kernel = pl.pallas_call(
    _combine_body,
    out_shape=jax.ShapeDtypeStruct((NPAD, F), jnp.float32),
)


def kernel(x, edge_index, W1, b1, W2, b2):
    E = edge_index.shape[1]
    x = x.astype(jnp.float32)
    src = edge_index[0].astype(jnp.int32)
    dst = edge_index[1].astype(jnp.int32)
    pad = EPAD - E
    if pad >= 0:
        src = jnp.concatenate([src, jnp.zeros((pad,), jnp.int32)])
        dst = jnp.concatenate([dst, jnp.full((pad,), N, jnp.int32)])
    else:
        src = src[:EPAD]
        dst = dst[:EPAD]
    src3 = src.reshape(NTILES, CPT, CHUNK)
    dst3 = dst.reshape(NTILES, CPT, CHUNK)
    xp = jnp.pad(x, ((0, NPAD - N), (0, 0)))
    zeros = jnp.zeros((NPAD, F), jnp.float32)
    ones = jnp.ones((NPAD, F), jnp.float32)

    d0, d1 = _edge_kernel(ones, src3, dst3, zeros)
    s0, cp, g, c2, g2 = _prep_kernel(
        xp, W1.astype(jnp.float32), b1.astype(jnp.float32).reshape(1, -1),
        W2.astype(jnp.float32), b2.astype(jnp.float32).reshape(1, -1),
        d0, d1)

    s = s0
    for _ in range(K_STEPS - 1):
        t0, t1 = _edge_kernel(s, src3, dst3, zeros)
        s = _combine_kernel(t0, t1, s, cp, g)
    t0, t1 = _edge_kernel(s, src3, dst3, zeros)
    out = _combine_kernel(t0, t1, s, c2, g2)
    return out[:N].astype(jnp.float64)


# speedup vs baseline: 1.3345x; 1.3345x over previous
import jax
import jax.numpy as jnp
from jax import lax
from jax.experimental import pallas as pl
from jax.experimental.pallas import tpu as pltpu
from jax.experimental.pallas import tpu_sc as plsc

N = 10000
NPAD = 10016
F = 32
ALPHA = 0.1
K_STEPS = 10
NCORES = 2
NSUB = 16
NTILES = NCORES * NSUB
CHUNK = 128
CPT = 79
EPT = CPT * CHUNK
EPAD = EPT * NTILES
ROWS_PT = NPAD // NSUB
NR = NPAD * F // 128

_sc_mesh = plsc.VectorSubcoreMesh(core_axis_name="c", subcore_axis_name="s")


NBUF = 8


def _edge_body(s_hbm, src_hbm, dst_hbm, zeros_hbm, t0_hbm, t1_hbm,
               srcv, dstv, *scr):
    rows = scr[:NBUF]
    tsh = scr[NBUF]
    ssh = scr[NBUF + 1]
    gsems = scr[NBUF + 2:2 * NBUF + 2]
    ssems = scr[2 * NBUF + 2:3 * NBUF + 2]
    zsem = scr[3 * NBUF + 2]
    cid = lax.axis_index("c")
    sid = lax.axis_index("s")
    wid = sid * NCORES + cid
    base = sid * jnp.int32(ROWS_PT)

    zd = pltpu.async_copy(zeros_hbm.at[pl.ds(base, ROWS_PT)],
                          tsh.at[pl.ds(base, ROWS_PT)], zsem)
    pltpu.sync_copy(s_hbm.at[pl.ds(base, ROWS_PT)],
                    ssh.at[pl.ds(base, ROWS_PT)])
    pltpu.sync_copy(src_hbm.at[wid], srcv)
    pltpu.sync_copy(dst_hbm.at[wid], dstv)
    zd.wait()
    plsc.subcore_barrier()

    gdesc = [None] * CPT
    sdesc = [None] * CPT

    def _gather(k):
        gdesc[k] = pltpu.async_copy(
            ssh.at[srcv.at[jnp.int32(k)]], rows[k % NBUF],
            gsems[k % NBUF])

    D = 6
    for k in range(min(D, CPT)):
        _gather(k)
    for j in range(CPT):
        jp = j + D
        if jp < CPT:
            if jp >= NBUF:
                sdesc[jp - NBUF].wait()
            _gather(jp)
        b = j % NBUF
        gdesc[j].wait()
        sdesc[j] = pltpu.async_copy(
            rows[b], tsh.at[dstv.at[jnp.int32(j)]], ssems[b], add=True)
    for j in range(max(CPT - NBUF, 0), CPT):
        sdesc[j].wait()

    plsc.subcore_barrier()

    @pl.when(cid == 0)
    def _w0():
        pltpu.sync_copy(tsh.at[pl.ds(base, ROWS_PT)],
                        t0_hbm.at[pl.ds(base, ROWS_PT)])

    @pl.when(cid == 1)
    def _w1():
        pltpu.sync_copy(tsh.at[pl.ds(base, ROWS_PT)],
                        t1_hbm.at[pl.ds(base, ROWS_PT)])


_edge_kernel = pl.kernel(
    _edge_body,
    out_type=(
        jax.ShapeDtypeStruct((NPAD, F), jnp.float32),
        jax.ShapeDtypeStruct((NPAD, F), jnp.float32),
    ),
    mesh=_sc_mesh,
    compiler_params=pltpu.CompilerParams(use_tc_tiling_on_sc=False),
    scratch_types=(
        [pltpu.VMEM((CPT, CHUNK), jnp.int32),
         pltpu.VMEM((CPT, CHUNK), jnp.int32)]
        + [pltpu.VMEM((CHUNK, F), jnp.float32) for _ in range(NBUF)]
        + [pltpu.VMEM_SHARED((NPAD, F), jnp.float32)]
        + [pltpu.VMEM_SHARED((NPAD, F), jnp.float32)]
        + [pltpu.SemaphoreType.DMA for _ in range(2 * NBUF + 1)]
    ),
)


def _prep_body(x_ref, w1_ref, b1_ref, w2_ref, b2_ref, d0_ref, d1_ref,
               s0_ref, cp_ref, g_ref, c2_ref, g2_ref):
    h1 = jnp.dot(x_ref[...], w1_ref[...], preferred_element_type=jnp.float32)
    h1 = jnp.maximum(h1 + b1_ref[...], 0.0)
    h = jnp.dot(h1, w2_ref[...], preferred_element_type=jnp.float32)
    h = h + b2_ref[...]
    deg = d0_ref[...] + d1_ref[...] + 1.0
    dinv = lax.rsqrt(deg)
    s0 = dinv * h
    s0_ref[...] = s0
    cp_ref[...] = (1.0 - ALPHA) * dinv * dinv
    g_ref[...] = ALPHA * s0
    c2_ref[...] = (1.0 - ALPHA) * dinv
    g2_ref[...] = ALPHA * h


_prep_kernel = pl.pallas_call(
    _prep_body,
    out_shape=tuple(
        jax.ShapeDtypeStruct((NPAD, F), jnp.float32) for _ in range(5)
    ),
)


def _combine_body(t0_ref, t1_ref, s_ref, c_ref, g_ref, out_ref):
    out_ref[...] = (
        c_ref[...] * ((t0_ref[...] + t1_ref[...]) + s_ref[...]) + g_ref[...]
    )


_combine_kernel = pl.pallas_call(
    _combine_body,
    out_shape=jax.ShapeDtypeStruct((NR, 128), jnp.float32),
)


def kernel(x, edge_index, W1, b1, W2, b2):
    E = edge_index.shape[1]
    x = x.astype(jnp.float32)
    src = edge_index[0].astype(jnp.int32)
    dst = edge_index[1].astype(jnp.int32)
    pad = EPAD - E
    if pad >= 0:
        src = jnp.concatenate([src, jnp.zeros((pad,), jnp.int32)])
        dst = jnp.concatenate([dst, jnp.full((pad,), N, jnp.int32)])
    else:
        src = src[:EPAD]
        dst = dst[:EPAD]
    src3 = src.reshape(NTILES, CPT, CHUNK)
    dst3 = dst.reshape(NTILES, CPT, CHUNK)
    xp = jnp.pad(x, ((0, NPAD - N), (0, 0)))
    zeros = jnp.zeros((NPAD, F), jnp.float32)
    ones = jnp.ones((NPAD, F), jnp.float32)

    d0, d1 = _edge_kernel(ones, src3, dst3, zeros)
    s0, cp, g, c2, g2 = _prep_kernel(
        xp, W1.astype(jnp.float32), b1.astype(jnp.float32).reshape(1, -1),
        W2.astype(jnp.float32), b2.astype(jnp.float32).reshape(1, -1),
        d0, d1)

    cpv = cp.reshape(NR, 128)
    gv = g.reshape(NR, 128)
    s = s0
    for _ in range(K_STEPS - 1):
        t0, t1 = _edge_kernel(s, src3, dst3, zeros)
        s = _combine_kernel(
            t0.reshape(NR, 128), t1.reshape(NR, 128), s.reshape(NR, 128),
            cpv, gv).reshape(NPAD, F)
    t0, t1 = _edge_kernel(s, src3, dst3, zeros)
    out = _combine_kernel(
        t0.reshape(NR, 128), t1.reshape(NR, 128), s.reshape(NR, 128),
        c2.reshape(NR, 128), g2.reshape(NR, 128))
    return out.reshape(NPAD, F)[:N].astype(jnp.float64)
